# baseline (speedup 1.0000x reference)
import jax
import jax.numpy as jnp
from jax.experimental import pallas as pl


_N = 512
_K = 1024
_D = 256


def _tree_reduce_cols(sqt):
    halves = []
    for t in (0, 1):
        base = 128 * t
        acc = sqt[base:base + 8, :]
        for v in range(1, 16):
            acc = acc + sqt[base + 8 * v:base + 8 * v + 8, :]
        p = acc
        r = ((p[0:1, :] + p[4:5, :]) + (p[2:3, :] + p[6:7, :])) + (
            (p[1:2, :] + p[5:6, :]) + (p[3:4, :] + p[7:8, :]))
        halves.append(r)
    return halves[0] + halves[1]


def _vq_kernel(z_ref, w_ref, idx_ref, zq_ref, ma_ref):
    w = w_ref[...]

    def body(i, _):
        zrow = z_ref[pl.ds(i, 1), :]
        diff = zrow - w
        sq = diff * diff
        sqt = jnp.swapaxes(sq, 0, 1)
        d = _tree_reduce_cols(sqt)
        dmin = jnp.min(d)
        iota = jax.lax.broadcasted_iota(jnp.int32, (1, _K), 1)
        cand = jnp.where(d == dmin, iota, _K)
        idx = jnp.min(cand)
        idx_ref[pl.ds(i, 1), :] = jnp.full((1, 1), idx, jnp.int32)
        zqrow = w_ref[pl.ds(idx, 1), :]
        zq_ref[pl.ds(i, 1), :] = zqrow
        ma_ref[pl.ds(i, 1), :] = zrow + (zqrow - zrow)
        return 0

    jax.lax.fori_loop(0, _N, body, 0)


def kernel(x, weight):
    z = jnp.transpose(x, (0, 2, 3, 1))
    zf = z.reshape(_N, _D)
    idx2, zqf, maf = pl.pallas_call(
        _vq_kernel,
        out_shape=(
            jax.ShapeDtypeStruct((_N, 1), jnp.int32),
            jax.ShapeDtypeStruct((_N, _D), jnp.float32),
            jax.ShapeDtypeStruct((_N, _D), jnp.float32),
        ),
    )(zf, weight)
    indices = idx2.reshape(_N)
    z_q = zqf.reshape(z.shape)
    z_q_ma = jnp.transpose(maf.reshape(z.shape), (0, 3, 1, 2))
    return (z_q_ma, z_q, z, indices)


# speedup vs baseline: 20.1230x; 20.1230x over previous
import jax
import jax.numpy as jnp
from jax.experimental import pallas as pl


_N = 512
_K = 1024
_D = 256
_M = 4

_HI = jax.lax.Precision.HIGHEST


def _tree_reduce_cols(sqt):
    halves = []
    for t in (0, 1):
        base = 128 * t
        acc = sqt[base:base + 8, :]
        for v in range(1, 16):
            acc = acc + sqt[base + 8 * v:base + 8 * v + 8, :]
        p = acc
        r = ((p[0:1, :] + p[4:5, :]) + (p[2:3, :] + p[6:7, :])) + (
            (p[1:2, :] + p[5:6, :]) + (p[3:4, :] + p[7:8, :]))
        halves.append(r)
    return halves[0] + halves[1]


def _vq_kernel(z_ref, w_ref, idx_ref, zq_ref, ma_ref):
    z = z_ref[...]
    w = w_ref[...]

    wn = jnp.sum(w * w, axis=1, keepdims=True)
    wn_row = jnp.swapaxes(wn, 0, 1)
    scores = jax.lax.dot_general(z, w, (((1,), (1,)), ((), ())),
                                 preferred_element_type=jnp.float32,
                                 precision=_HI)
    da = wn_row - 2.0 * scores

    iota_k = jax.lax.broadcasted_iota(jnp.int32, (_N, _K), 1)
    cand_cols = []
    for j in range(_M):
        dmin = jnp.min(da, axis=1, keepdims=True)
        idx_j = jnp.min(jnp.where(da == dmin, iota_k, _K),
                        axis=1, keepdims=True)
        cand_cols.append(idx_j)
        if j < _M - 1:
            da = jnp.where(iota_k == idx_j, jnp.float32(3e38), da)

    idx_cat = jnp.concatenate(cand_cols, axis=0)

    onehot = (jax.lax.broadcasted_iota(jnp.int32, (_M * _N, _K), 1)
              == idx_cat).astype(jnp.float32)
    gt = jax.lax.dot_general(w, onehot, (((0,), (1,)), ((), ())),
                             preferred_element_type=jnp.float32,
                             precision=_HI)

    zt = jnp.swapaxes(z, 0, 1)
    zt_rep = jnp.concatenate([zt] * _M, axis=1)
    diff = zt_rep - gt
    dx = _tree_reduce_cols(diff * diff)

    best_d = dx[:, 0:_N]
    best_i = jnp.swapaxes(cand_cols[0], 0, 1)
    best_g = gt[:, 0:_N]
    for j in range(1, _M):
        dj = dx[:, j * _N:(j + 1) * _N]
        ij = jnp.swapaxes(cand_cols[j], 0, 1)
        better = (dj < best_d) | ((dj == best_d) & (ij < best_i))
        best_d = jnp.where(better, dj, best_d)
        best_i = jnp.where(better, ij, best_i)
        best_g = jnp.where(better, gt[:, j * _N:(j + 1) * _N], best_g)

    idx_ref[...] = jnp.swapaxes(best_i, 0, 1)
    zq = jnp.swapaxes(best_g, 0, 1)
    zq_ref[...] = zq
    ma_ref[...] = z + (zq - z)


def kernel(x, weight):
    z = jnp.transpose(x, (0, 2, 3, 1))
    zf = z.reshape(_N, _D)
    idx2, zqf, maf = pl.pallas_call(
        _vq_kernel,
        out_shape=(
            jax.ShapeDtypeStruct((_N, 1), jnp.int32),
            jax.ShapeDtypeStruct((_N, _D), jnp.float32),
            jax.ShapeDtypeStruct((_N, _D), jnp.float32),
        ),
    )(zf, weight)
    indices = idx2.reshape(_N)
    z_q = zqf.reshape(z.shape)
    z_q_ma = jnp.transpose(maf.reshape(z.shape), (0, 3, 1, 2))
    return (z_q_ma, z_q, z, indices)
